# baseline (speedup 1.0000x reference)
import functools
import math

import jax
import jax.numpy as jnp
from jax import lax
from jax.experimental import pallas as pl
from jax.experimental.pallas import tpu as pltpu
from jax.experimental.pallas import tpu_sc as plsc

D = 128
SCALE = math.sqrt(128.0)
NW = 32
RPC = 2


@functools.lru_cache(maxsize=None)
def _make_kernel(B: int, lp: int):
    rows_per_w = B // NW
    n_chunks = rows_per_w // RPC
    kc = RPC * lp
    assert n_chunks >= 4 and n_chunks % 2 == 0 and kc <= 128
    mesh = plsc.VectorSubcoreMesh(core_axis_name="c", subcore_axis_name="s")

    @functools.partial(
        pl.kernel,
        out_type=jax.ShapeDtypeStruct((B, lp, D), jnp.float32),
        mesh=mesh,
        compiler_params=pltpu.CompilerParams(use_tc_tiling_on_sc=False),
        scratch_types=[
            pltpu.VMEM((n_chunks, kc), jnp.int32),
            pltpu.VMEM((2, kc, D), jnp.float32),
            pltpu.VMEM((2, kc, D), jnp.float32),
            pltpu.SemaphoreType.DMA,
            pltpu.SemaphoreType.DMA,
            pltpu.SemaphoreType.DMA,
            pltpu.SemaphoreType.DMA,
        ],
    )
    def emb(idx_hbm, table_hbm, out_hbm, idx_v, gbuf, sbuf,
            gsem0, gsem1, ssem0, ssem1):
        wid = lax.axis_index("s") * 2 + lax.axis_index("c")
        pltpu.sync_copy(idx_hbm.at[wid], idx_v)
        base = wid * rows_per_w
        gsems = (gsem0, gsem1)
        ssems = (ssem0, ssem1)

        def fire_gather(b, j):
            pltpu.async_copy(table_hbm.at[idx_v.at[j]], gbuf.at[b], gsems[b])

        def wait_gather(b):
            pltpu.make_async_copy(
                table_hbm.at[pl.ds(0, kc)], gbuf.at[b], gsems[b]).wait()

        def fire_store(b, j):
            for r in range(RPC):
                pltpu.async_copy(
                    sbuf.at[b, pl.ds(r * lp, lp)],
                    out_hbm.at[base + j * RPC + r], ssems[b])

        def wait_store(b):
            for r in range(RPC):
                pltpu.make_async_copy(
                    sbuf.at[b, pl.ds(r * lp, lp)],
                    out_hbm.at[0], ssems[b]).wait()

        def scale(b):
            def row(i, c):
                for c8 in range(D // 16):
                    s = pl.ds(c8 * 16, 16)
                    sbuf[b, i, s] = gbuf[b, i, s] * SCALE
                return c
            lax.fori_loop(0, kc, row, 0)

        fire_gather(0, 0)
        fire_gather(1, 1)
        for j in range(2):
            b = j % 2
            wait_gather(b)
            scale(b)
            fire_store(b, j)
            fire_gather(b, j + 2)

        def group(j2, c):
            for b in range(2):
                j = 2 * j2 + b
                wait_gather(b)
                wait_store(b)
                scale(b)
                fire_store(b, j)
                fire_gather(b, j + 2)
            return c

        lax.fori_loop(1, n_chunks // 2 - 1, group, 0)

        for j in range(n_chunks - 2, n_chunks):
            b = j % 2
            wait_gather(b)
            wait_store(b)
            scale(b)
            fire_store(b, j)
        wait_store(0)
        wait_store(1)

    return emb


def kernel(x, table):
    B, L = x.shape
    lp = (L + 7) // 8 * 8
    idx = jnp.pad(x.astype(jnp.int32), ((0, 0), (0, lp - L)))
    idx = idx.reshape(NW, (B // NW) // RPC, RPC * lp)
    out = _make_kernel(B, lp)(idx, table)
    return out[:, :L, :]


# speedup vs baseline: 5.9191x; 5.9191x over previous
import functools
import math

import jax
import jax.numpy as jnp
from jax import lax
from jax.experimental import pallas as pl
from jax.experimental.pallas import tpu as pltpu
from jax.experimental.pallas import tpu_sc as plsc

D = 128
SCALE = math.sqrt(128.0)
NW = 32
RPC = 2


@functools.lru_cache(maxsize=None)
def _make_kernel(B: int, lp: int):
    rows_per_w = B // NW
    n_chunks = rows_per_w // RPC
    kc = RPC * lp
    assert n_chunks >= 4 and n_chunks % 2 == 0 and kc <= 128
    mesh = plsc.VectorSubcoreMesh(core_axis_name="c", subcore_axis_name="s")

    @functools.partial(
        pl.kernel,
        out_type=jax.ShapeDtypeStruct((B, lp, D), jnp.float32),
        mesh=mesh,
        compiler_params=pltpu.CompilerParams(use_tc_tiling_on_sc=False),
        scratch_types=[
            pltpu.VMEM((n_chunks, kc), jnp.int32),
            pltpu.VMEM((2, kc, D), jnp.float32),
            pltpu.VMEM((2, kc, D), jnp.float32),
            pltpu.SemaphoreType.DMA,
            pltpu.SemaphoreType.DMA,
            pltpu.SemaphoreType.DMA,
            pltpu.SemaphoreType.DMA,
        ],
    )
    def emb(idx_hbm, table_hbm, out_hbm, idx_v, gbuf, sbuf,
            gsem0, gsem1, ssem0, ssem1):
        wid = lax.axis_index("s") * 2 + lax.axis_index("c")
        pltpu.sync_copy(idx_hbm.at[wid], idx_v)
        base = wid * rows_per_w
        gsems = (gsem0, gsem1)
        ssems = (ssem0, ssem1)

        def fire_gather(b, j):
            pltpu.async_copy(table_hbm.at[idx_v.at[j]], gbuf.at[b], gsems[b])

        def wait_gather(b):
            pltpu.make_async_copy(
                table_hbm.at[pl.ds(0, kc)], gbuf.at[b], gsems[b]).wait()

        def fire_store(b, j):
            for r in range(RPC):
                pltpu.async_copy(
                    sbuf.at[b, pl.ds(r * lp, lp)],
                    out_hbm.at[base + j * RPC + r], ssems[b])

        def wait_store(b):
            for r in range(RPC):
                pltpu.make_async_copy(
                    sbuf.at[b, pl.ds(r * lp, lp)],
                    out_hbm.at[0], ssems[b]).wait()

        def scale(b):
            def row(i, c):
                for c8 in range(D // 16):
                    s = pl.ds(c8 * 16, 16)
                    sbuf[b, i, s] = gbuf[b, i, s] * SCALE
                return c
            lax.fori_loop(0, kc, row, 0)

        fire_gather(0, 0)
        fire_gather(1, 1)
        for j in range(2):
            b = j % 2
            wait_gather(b)
            scale(b)
            fire_store(b, j)
            fire_gather(b, j + 2)

        def group(j2, c):
            for b in range(2):
                j = 2 * j2 + b
                wait_gather(b)
                wait_store(b)
                scale(b)
                fire_store(b, j)
                fire_gather(b, j + 2)
            return c

        lax.fori_loop(1, n_chunks // 2 - 1, group, 0)

        for j in range(n_chunks - 2, n_chunks):
            b = j % 2
            wait_gather(b)
            wait_store(b)
            scale(b)
            fire_store(b, j)
        wait_store(0)
        wait_store(1)

    return emb


def kernel(x, table):
    B, L = x.shape
    lp = (L + 7) // 8 * 8
    idx = jnp.pad(x.astype(jnp.int32), ((0, 0), (0, lp - L)), mode="edge")
    idx = idx.reshape(NW, (B // NW) // RPC, RPC * lp)
    out = _make_kernel(B, lp)(idx, table)
    return out[:, :L, :]


# speedup vs baseline: 5.9333x; 1.0024x over previous
import functools
import math

import jax
import jax.numpy as jnp
from jax import lax
from jax.experimental import pallas as pl
from jax.experimental.pallas import tpu as pltpu
from jax.experimental.pallas import tpu_sc as plsc

D = 128
SCALE = math.sqrt(128.0)
NW = 32
RPC = 2


@functools.lru_cache(maxsize=None)
def _make_kernel(B: int, lp: int):
    rows_per_w = B // NW
    n_chunks = rows_per_w // RPC
    kc = RPC * lp
    assert n_chunks >= 4 and n_chunks % 2 == 0 and kc <= 128
    mesh = plsc.VectorSubcoreMesh(core_axis_name="c", subcore_axis_name="s")

    @functools.partial(
        pl.kernel,
        out_type=jax.ShapeDtypeStruct((B, lp, D), jnp.float32),
        mesh=mesh,
        scratch_types=[
            pltpu.VMEM((n_chunks * kc,), jnp.int32),
            pltpu.VMEM((2, kc, D), jnp.float32),
            pltpu.VMEM((2, kc, D), jnp.float32),
            pltpu.SemaphoreType.DMA,
            pltpu.SemaphoreType.DMA,
            pltpu.SemaphoreType.DMA,
            pltpu.SemaphoreType.DMA,
        ],
    )
    def emb(idx_hbm, table_hbm, out_hbm, idx_v, gbuf, sbuf,
            gsem0, gsem1, ssem0, ssem1):
        wid = lax.axis_index("s") * 2 + lax.axis_index("c")
        pltpu.sync_copy(idx_hbm.at[wid], idx_v)
        base = wid * rows_per_w
        gsems = (gsem0, gsem1)
        ssems = (ssem0, ssem1)

        def fire_gather(b, j):
            pltpu.async_copy(
                table_hbm.at[idx_v.at[pl.ds(j * kc, kc)]],
                gbuf.at[b], gsems[b])

        def wait_gather(b):
            pltpu.make_async_copy(
                table_hbm.at[pl.ds(0, kc)], gbuf.at[b], gsems[b]).wait()

        def fire_store(b, j):
            for r in range(RPC):
                pltpu.async_copy(
                    sbuf.at[b, pl.ds(r * lp, lp)],
                    out_hbm.at[base + j * RPC + r], ssems[b])

        def wait_store(b):
            for r in range(RPC):
                pltpu.make_async_copy(
                    sbuf.at[b, pl.ds(r * lp, lp)],
                    out_hbm.at[0], ssems[b]).wait()

        def scale(b):
            def row(i, c):
                for c8 in range(D // 16):
                    s = pl.ds(c8 * 16, 16)
                    sbuf[b, i, s] = gbuf[b, i, s] * SCALE
                return c
            lax.fori_loop(0, kc, row, 0)

        fire_gather(0, 0)
        fire_gather(1, 1)
        for j in range(2):
            b = j % 2
            wait_gather(b)
            scale(b)
            fire_store(b, j)
            fire_gather(b, j + 2)

        def group(j2, c):
            for b in range(2):
                j = 2 * j2 + b
                wait_gather(b)
                wait_store(b)
                scale(b)
                fire_store(b, j)
                fire_gather(b, j + 2)
            return c

        lax.fori_loop(1, n_chunks // 2 - 1, group, 0)

        for j in range(n_chunks - 2, n_chunks):
            b = j % 2
            wait_gather(b)
            wait_store(b)
            scale(b)
            fire_store(b, j)
        wait_store(0)
        wait_store(1)

    return emb


def kernel(x, table):
    B, L = x.shape
    lp = (L + 7) // 8 * 8
    idx = jnp.pad(x.astype(jnp.int32), ((0, 0), (0, lp - L)), mode="edge")
    idx = idx.reshape(NW, (B // NW) * lp)
    out = _make_kernel(B, lp)(idx, table)
    return out[:, :L, :]


# speedup vs baseline: 6.6022x; 1.1127x over previous
import functools
import math

import jax
import jax.numpy as jnp
from jax import lax
from jax.experimental import pallas as pl
from jax.experimental.pallas import tpu as pltpu
from jax.experimental.pallas import tpu_sc as plsc

D = 128
SCALE = math.sqrt(128.0)
NW = 32
RPC = 2


@functools.lru_cache(maxsize=None)
def _make_kernel(B: int, L: int):
    lp = (L + 7) // 8 * 8
    rows_per_w = B // NW
    n_chunks = rows_per_w // RPC
    kc = RPC * L
    assert n_chunks >= 4 and n_chunks % 2 == 0 and kc <= 128
    mesh = plsc.VectorSubcoreMesh(core_axis_name="c", subcore_axis_name="s")

    @functools.partial(
        pl.kernel,
        out_type=jax.ShapeDtypeStruct((B, lp, D), jnp.float32),
        mesh=mesh,
        compiler_params=pltpu.CompilerParams(use_tc_tiling_on_sc=False),
        scratch_types=[
            pltpu.VMEM((n_chunks, kc), jnp.int32),
            pltpu.VMEM((2, kc, D), jnp.float32),
            pltpu.VMEM((2, RPC * lp, D), jnp.float32),
            pltpu.SemaphoreType.DMA,
            pltpu.SemaphoreType.DMA,
            pltpu.SemaphoreType.DMA,
            pltpu.SemaphoreType.DMA,
        ],
    )
    def emb(idx_hbm, table_hbm, out_hbm, idx_v, gbuf, sbuf,
            gsem0, gsem1, ssem0, ssem1):
        wid = lax.axis_index("s") * 2 + lax.axis_index("c")
        pltpu.sync_copy(idx_hbm.at[wid], idx_v)
        base = wid * rows_per_w
        gsems = (gsem0, gsem1)
        ssems = (ssem0, ssem1)

        def fire_gather(b, j):
            pltpu.async_copy(table_hbm.at[idx_v.at[j]], gbuf.at[b], gsems[b])

        def wait_gather(b):
            pltpu.make_async_copy(
                table_hbm.at[pl.ds(0, kc)], gbuf.at[b], gsems[b]).wait()

        def fire_store(b, j):
            for r in range(RPC):
                pltpu.async_copy(
                    sbuf.at[b, pl.ds(r * lp, lp)],
                    out_hbm.at[base + j * RPC + r], ssems[b])

        def wait_store(b):
            for r in range(RPC):
                pltpu.make_async_copy(
                    sbuf.at[b, pl.ds(r * lp, lp)],
                    out_hbm.at[0], ssems[b]).wait()

        def scale(b):
            def row(i, c):
                for r in range(RPC):
                    for c8 in range(D // 16):
                        s = pl.ds(c8 * 16, 16)
                        sbuf[b, r * lp + i, s] = gbuf[b, r * L + i, s] * SCALE
                return c
            lax.fori_loop(0, L, row, 0)

        fire_gather(0, 0)
        fire_gather(1, 1)
        for j in range(2):
            b = j % 2
            wait_gather(b)
            scale(b)
            fire_store(b, j)
            fire_gather(b, j + 2)

        def group(j2, c):
            for b in range(2):
                j = 2 * j2 + b
                wait_gather(b)
                wait_store(b)
                scale(b)
                fire_store(b, j)
                fire_gather(b, j + 2)
            return c

        lax.fori_loop(1, n_chunks // 2 - 1, group, 0)

        for j in range(n_chunks - 2, n_chunks):
            b = j % 2
            wait_gather(b)
            wait_store(b)
            scale(b)
            fire_store(b, j)
        wait_store(0)
        wait_store(1)

    return emb


def kernel(x, table):
    B, L = x.shape
    idx = x.reshape(NW, (B // NW) // RPC, RPC * L).astype(jnp.int32)
    out = _make_kernel(B, L)(idx, table)
    return out[:, :L, :]


# speedup vs baseline: 6.6271x; 1.0038x over previous
import functools
import math

import jax
import jax.numpy as jnp
from jax import lax
from jax.experimental import pallas as pl
from jax.experimental.pallas import tpu as pltpu
from jax.experimental.pallas import tpu_sc as plsc

D = 128
SCALE = math.sqrt(128.0)
NW = 32
RPC = 2


@functools.lru_cache(maxsize=None)
def _make_kernel(B: int, L: int):
    lp = (L + 7) // 8 * 8
    rows_per_w = B // NW
    n_chunks = rows_per_w // RPC
    kc = RPC * L
    assert n_chunks >= 4 and n_chunks % 2 == 0 and kc <= 128
    mesh = plsc.VectorSubcoreMesh(core_axis_name="c", subcore_axis_name="s")

    @functools.partial(
        pl.kernel,
        out_type=jax.ShapeDtypeStruct((B, lp, D), jnp.float32),
        mesh=mesh,
        compiler_params=pltpu.CompilerParams(use_tc_tiling_on_sc=False),
        scratch_types=[
            pltpu.VMEM((n_chunks, kc), jnp.int32),
            pltpu.VMEM((2, kc, D), jnp.float32),
            pltpu.VMEM((2, RPC * lp, D), jnp.float32),
            pltpu.SemaphoreType.DMA,
            pltpu.SemaphoreType.DMA,
            pltpu.SemaphoreType.DMA,
            pltpu.SemaphoreType.DMA,
        ],
    )
    def emb(idx_hbm, table_hbm, out_hbm, idx_v, gbuf, sbuf,
            gsem0, gsem1, ssem0, ssem1):
        wid = lax.axis_index("s") * 2 + lax.axis_index("c")
        pltpu.sync_copy(idx_hbm.at[wid], idx_v)
        base = wid * rows_per_w
        gsems = (gsem0, gsem1)
        ssems = (ssem0, ssem1)

        def fire_gather(b, j):
            pltpu.async_copy(table_hbm.at[idx_v.at[j]], gbuf.at[b], gsems[b])

        def wait_gather(b, j):
            pltpu.make_async_copy(
                table_hbm.at[idx_v.at[j]], gbuf.at[b], gsems[b]).wait()

        def fire_store(b, j):
            for r in range(RPC):
                pltpu.async_copy(
                    sbuf.at[b, pl.ds(r * lp, lp)],
                    out_hbm.at[base + j * RPC + r], ssems[b])

        def wait_store(b):
            for r in range(RPC):
                pltpu.make_async_copy(
                    sbuf.at[b, pl.ds(r * lp, lp)],
                    out_hbm.at[0], ssems[b]).wait()

        def scale(b):
            def row(i, c):
                for r in range(RPC):
                    for c8 in range(D // 16):
                        s = pl.ds(c8 * 16, 16)
                        sbuf[b, r * lp + i, s] = gbuf[b, r * L + i, s] * SCALE
                return c
            lax.fori_loop(0, L, row, 0)

        fire_gather(0, 0)
        fire_gather(1, 1)
        for j in range(2):
            b = j % 2
            wait_gather(b, j)
            scale(b)
            fire_store(b, j)
            fire_gather(b, j + 2)

        def group(j2, c):
            for b in range(2):
                j = 2 * j2 + b
                wait_gather(b, j)
                wait_store(b)
                scale(b)
                fire_store(b, j)
                fire_gather(b, j + 2)
            return c

        lax.fori_loop(1, n_chunks // 2 - 1, group, 0)

        for j in range(n_chunks - 2, n_chunks):
            b = j % 2
            wait_gather(b, j)
            wait_store(b)
            scale(b)
            fire_store(b, j)
        wait_store(0)
        wait_store(1)

    return emb


def kernel(x, table):
    B, L = x.shape
    idx = x.reshape(NW, (B // NW) // RPC, RPC * L).astype(jnp.int32)
    out = _make_kernel(B, L)(idx, table)
    return out[:, :L, :]


# speedup vs baseline: 6.7232x; 1.0145x over previous
import functools
import math

import jax
import jax.numpy as jnp
from jax import lax
from jax.experimental import pallas as pl
from jax.experimental.pallas import tpu as pltpu
from jax.experimental.pallas import tpu_sc as plsc

D = 128
SCALE = math.sqrt(128.0)
NW = 32
RPC = 2


@functools.lru_cache(maxsize=None)
def _make_kernel(B: int, L: int):
    lp = (L + 7) // 8 * 8
    rows_per_w = B // NW
    n_chunks = rows_per_w // RPC
    kc = RPC * L
    assert n_chunks >= 4 and n_chunks % 2 == 0 and kc <= 128
    mesh = plsc.VectorSubcoreMesh(core_axis_name="c", subcore_axis_name="s")

    @functools.partial(
        pl.kernel,
        out_type=jax.ShapeDtypeStruct((B, lp, D), jnp.float32),
        mesh=mesh,
        compiler_params=pltpu.CompilerParams(use_tc_tiling_on_sc=False),
        scratch_types=[
            pltpu.VMEM((n_chunks, kc), jnp.int32),
            pltpu.VMEM((2, kc, D), jnp.float32),
            pltpu.VMEM((2, RPC, L, D), jnp.float32),
            pltpu.SemaphoreType.DMA,
            pltpu.SemaphoreType.DMA,
            pltpu.SemaphoreType.DMA,
            pltpu.SemaphoreType.DMA,
        ],
    )
    def emb(idx_hbm, table_hbm, out_hbm, idx_v, gbuf, sbuf,
            gsem0, gsem1, ssem0, ssem1):
        wid = lax.axis_index("s") * 2 + lax.axis_index("c")
        pltpu.sync_copy(idx_hbm.at[wid], idx_v)
        base = wid * rows_per_w
        gsems = (gsem0, gsem1)
        ssems = (ssem0, ssem1)

        def fire_gather(b, j):
            pltpu.async_copy(table_hbm.at[idx_v.at[j]], gbuf.at[b], gsems[b])

        def wait_gather(b, j):
            pltpu.make_async_copy(
                table_hbm.at[idx_v.at[j]], gbuf.at[b], gsems[b]).wait()

        def fire_store(b, j):
            pltpu.async_copy(
                sbuf.at[b],
                out_hbm.at[pl.ds(base + j * RPC, RPC), pl.ds(0, L)], ssems[b])

        def wait_store(b):
            pltpu.make_async_copy(
                sbuf.at[b],
                out_hbm.at[pl.ds(0, RPC), pl.ds(0, L)], ssems[b]).wait()

        def scale(b):
            def row(i, c):
                for r in range(RPC):
                    for c8 in range(D // 16):
                        s = pl.ds(c8 * 16, 16)
                        sbuf[b, r, i, s] = gbuf[b, r * L + i, s] * SCALE
                return c
            lax.fori_loop(0, L, row, 0)

        fire_gather(0, 0)
        fire_gather(1, 1)
        for j in range(2):
            b = j % 2
            wait_gather(b, j)
            scale(b)
            fire_store(b, j)
            fire_gather(b, j + 2)

        def group(j2, c):
            for b in range(2):
                j = 2 * j2 + b
                wait_gather(b, j)
                wait_store(b)
                scale(b)
                fire_store(b, j)
                fire_gather(b, j + 2)
            return c

        lax.fori_loop(1, n_chunks // 2 - 1, group, 0)

        for j in range(n_chunks - 2, n_chunks):
            b = j % 2
            wait_gather(b, j)
            wait_store(b)
            scale(b)
            fire_store(b, j)
        wait_store(0)
        wait_store(1)

    return emb


def kernel(x, table):
    B, L = x.shape
    idx = x.reshape(NW, (B // NW) // RPC, RPC * L).astype(jnp.int32)
    out = _make_kernel(B, L)(idx, table)
    return out[:, :L, :]
